# Initial kernel scaffold; baseline (speedup 1.0000x reference)
import jax
import jax.numpy as jnp
from jax.experimental import pallas as pl


def kernel(head, relation, tail, entity_embedding, relation_embedding):
    raise NotImplementedError("write your pallas kernel here")


# speedup vs baseline: 2.3391x; 2.3391x over previous
import functools

import jax
import jax.numpy as jnp
from jax import lax
from jax.experimental import pallas as pl
from jax.experimental.pallas import tpu as pltpu
from jax.experimental.pallas import tpu_sc as plsc

_BATCH = 16384
_D = 128
_L = 16
_NC = 2
_NS = 16
_NW = _NC * _NS
_BPW = _BATCH // _NW
_C = 64
_NCHUNK = _BPW // _C
_NBUF = 2

_mesh = plsc.VectorSubcoreMesh(core_axis_name="c", subcore_axis_name="s")


@functools.partial(
    pl.kernel,
    mesh=_mesh,
    out_type=jax.ShapeDtypeStruct((_BATCH,), jnp.float32),
    scratch_types=[
        pltpu.VMEM((3, _NCHUNK, _C), jnp.int32),
        pltpu.VMEM((_NBUF, _C, _D), jnp.float32),
        pltpu.VMEM((_NBUF, _C, _D), jnp.float32),
        pltpu.VMEM((_NBUF, _C, _D), jnp.float32),
        pltpu.VMEM((_BPW,), jnp.float32),
        pltpu.SemaphoreType.DMA,
        pltpu.SemaphoreType.DMA,
        pltpu.SemaphoreType.DMA,
    ],
)
def _distmult_sc(head_hbm, rel_hbm, tail_hbm, ent_hbm, relemb_hbm, out_hbm,
                 idx, hbuf, rbuf, tbuf, scores, isem, gsem0, gsem1):
    wid = lax.axis_index("s") * _NC + lax.axis_index("c")
    base = wid * _BPW
    gsems = (gsem0, gsem1)

    staged = []
    for c in range(_NCHUNK):
        off = base + c * _C
        staged.append(pltpu.async_copy(head_hbm.at[pl.ds(off, _C)], idx.at[0, c], isem))
        staged.append(pltpu.async_copy(rel_hbm.at[pl.ds(off, _C)], idx.at[1, c], isem))
        staged.append(pltpu.async_copy(tail_hbm.at[pl.ds(off, _C)], idx.at[2, c], isem))
    for d in staged:
        d.wait()

    def fire(c, slot):
        return (
            pltpu.async_copy(ent_hbm.at[idx.at[0, c]], hbuf.at[slot], gsems[slot]),
            pltpu.async_copy(relemb_hbm.at[idx.at[1, c]], rbuf.at[slot], gsems[slot]),
            pltpu.async_copy(ent_hbm.at[idx.at[2, c]], tbuf.at[slot], gsems[slot]),
        )

    lane = lax.iota(jnp.int32, _L)
    lane0 = lane == 0

    dnums = lax.GatherDimensionNumbers(
        offset_dims=(), collapsed_slice_dims=(0,), start_index_map=(0,))

    def perm(v, idx):
        return lax.gather(v, idx[:, None], dnums, (1,),
                          mode=lax.GatherScatterMode.PROMISE_IN_BOUNDS)

    def xlane_sum(v):
        for sh in (8, 4, 2, 1):
            v = v + perm(v, lane ^ sh)
        return v

    def compute(c, slot):
        def row_body(i, vec):
            acc = (hbuf[slot, i, pl.ds(0, _L)]
                   * rbuf[slot, i, pl.ds(0, _L)]
                   * tbuf[slot, i, pl.ds(0, _L)])
            for j in range(1, _D // _L):
                acc = acc + (hbuf[slot, i, pl.ds(j * _L, _L)]
                             * rbuf[slot, i, pl.ds(j * _L, _L)]
                             * tbuf[slot, i, pl.ds(j * _L, _L)])
            total = xlane_sum(acc)
            vec = jnp.where(lane == (i & (_L - 1)), total, vec)

            @pl.when((i & (_L - 1)) == (_L - 1))
            def _store():
                scores[pl.ds(c * _C + i - (_L - 1), _L)] = vec

            return vec
        lax.fori_loop(0, _C, row_body, jnp.zeros((_L,), jnp.float32))

    pending = fire(0, 0)
    for c in range(_NCHUNK):
        nxt = fire(c + 1, (c + 1) % _NBUF) if c + 1 < _NCHUNK else None
        for d in pending:
            d.wait()
        compute(c, c % _NBUF)
        pending = nxt

    pltpu.sync_copy(scores, out_hbm.at[pl.ds(base, _BPW)])


def kernel(head, relation, tail, entity_embedding, relation_embedding):
    return _distmult_sc(head, relation, tail, entity_embedding, relation_embedding)
